# Initial kernel scaffold; baseline (speedup 1.0000x reference)
import jax
import jax.numpy as jnp
from jax.experimental import pallas as pl


def kernel(input_ids, position_ids, word_table, pos_table, type_table, gamma, beta):
    raise NotImplementedError("write your pallas kernel here")


# speedup vs baseline: 1.0965x; 1.0965x over previous
import functools

import jax
import jax.numpy as jnp
from jax import lax
from jax.experimental import pallas as pl
from jax.experimental.pallas import tpu as pltpu
from jax.experimental.pallas import tpu_sc as plsc

HID = 2048
EPS = 1e-05

NUM_CORES = 2
NUM_SUBCORES = 16
NUM_WORKERS = NUM_CORES * NUM_SUBCORES
LANES = 16
VECS_PER_ROW = HID // LANES

CHUNK = 16


def _make_gather_sum(num_tokens):
    tok_per_w = num_tokens // NUM_WORKERS
    n_chunks = tok_per_w // CHUNK
    mesh = plsc.VectorSubcoreMesh(
        core_axis_name="c", subcore_axis_name="s")

    @functools.partial(
        pl.kernel,
        out_type=jax.ShapeDtypeStruct((num_tokens, HID), jnp.float32),
        mesh=mesh,
        scratch_types=[
            pltpu.VMEM((tok_per_w,), jnp.int32),
            pltpu.VMEM((tok_per_w,), jnp.int32),
            pltpu.VMEM((CHUNK, HID), jnp.float32),
            pltpu.VMEM((CHUNK, HID), jnp.float32),
            pltpu.SemaphoreType.DMA,
            pltpu.SemaphoreType.DMA,
        ],
    )
    def gather_sum(ids_hbm, pids_hbm, wtab_hbm, ptab_hbm, out_hbm,
                   idx_v, pidx_v, wbuf, pbuf, sem_w, sem_p):
        wid = lax.axis_index("s") * NUM_CORES + lax.axis_index("c")
        base = wid * tok_per_w
        pltpu.sync_copy(ids_hbm.at[pl.ds(base, tok_per_w)], idx_v)
        pltpu.sync_copy(pids_hbm.at[pl.ds(base, tok_per_w)], pidx_v)

        def chunk_body(c, carry):
            off = c * CHUNK
            cw = pltpu.async_copy(
                wtab_hbm.at[idx_v.at[pl.ds(off, CHUNK)]], wbuf, sem_w)
            cp = pltpu.async_copy(
                ptab_hbm.at[pidx_v.at[pl.ds(off, CHUNK)]], pbuf, sem_p)
            cw.wait()
            cp.wait()

            def row_body(r, carry2):
                for v in range(VECS_PER_ROW):
                    sl = pl.ds(v * LANES, LANES)
                    wbuf[r, sl] = wbuf[r, sl] + pbuf[r, sl]
                return carry2

            lax.fori_loop(0, CHUNK, row_body, 0, unroll=False)
            pltpu.sync_copy(wbuf, out_hbm.at[pl.ds(base + off, CHUNK)])
            return carry

        lax.fori_loop(0, n_chunks, chunk_body, 0, unroll=False)

    return gather_sum


def _ln_body(x_ref, t_ref, g_ref, b_ref, o_ref):
    e = x_ref[...] + t_ref[...]
    mu = jnp.mean(e, axis=-1, keepdims=True)
    d = e - mu
    var = jnp.mean(d * d, axis=-1, keepdims=True)
    o_ref[...] = d * lax.rsqrt(var + EPS) * g_ref[...] + b_ref[...]


def _layernorm(summed, type_row, gamma, beta, blk):
    n = summed.shape[0]
    return pl.pallas_call(
        _ln_body,
        grid=(n // blk,),
        in_specs=[
            pl.BlockSpec((blk, HID), lambda i: (i, 0)),
            pl.BlockSpec((1, HID), lambda i: (0, 0)),
            pl.BlockSpec((1, HID), lambda i: (0, 0)),
            pl.BlockSpec((1, HID), lambda i: (0, 0)),
        ],
        out_specs=pl.BlockSpec((blk, HID), lambda i: (i, 0)),
        out_shape=jax.ShapeDtypeStruct((n, HID), jnp.float32),
    )(summed, type_row, gamma, beta)


def kernel(input_ids, position_ids, word_table, pos_table, type_table,
           gamma, beta):
    b, s = input_ids.shape
    n = b * s
    ids = input_ids.reshape(n)
    pids = position_ids.reshape(n)
    summed = _make_gather_sum(n)(ids, pids, word_table, pos_table)
    out = _layernorm(
        summed,
        type_table[0:1, :],
        gamma.reshape(1, HID),
        beta.reshape(1, HID),
        blk=512,
    )
    return out.reshape(b, s, HID)


# speedup vs baseline: 1.6400x; 1.4956x over previous
import functools

import jax
import jax.numpy as jnp
from jax import lax
from jax.experimental import pallas as pl
from jax.experimental.pallas import tpu as pltpu
from jax.experimental.pallas import tpu_sc as plsc

HID = 2048
EPS = 1e-05

NUM_CORES = 2
NUM_SUBCORES = 16
NUM_WORKERS = NUM_CORES * NUM_SUBCORES
LANES = 16
VECS_PER_ROW = HID // LANES

CHUNK = 8
NBUF = 2


def _make_gather_sum(num_tokens):
    tok_per_w = num_tokens // NUM_WORKERS
    n_chunks = tok_per_w // CHUNK
    n_outer = n_chunks // NBUF
    mesh = plsc.VectorSubcoreMesh(
        core_axis_name="c", subcore_axis_name="s")

    @functools.partial(
        pl.kernel,
        out_type=jax.ShapeDtypeStruct((num_tokens, HID), jnp.float32),
        mesh=mesh,
        scratch_types=[
            pltpu.VMEM((tok_per_w,), jnp.int32),
            pltpu.VMEM((tok_per_w,), jnp.int32),
            pltpu.VMEM((NBUF, CHUNK, HID), jnp.float32),
            pltpu.VMEM((NBUF, CHUNK, HID), jnp.float32),
            pltpu.VMEM((NBUF, CHUNK, HID), jnp.float32),
            [pltpu.SemaphoreType.DMA] * NBUF,
            [pltpu.SemaphoreType.DMA] * NBUF,
            [pltpu.SemaphoreType.DMA] * NBUF,
        ],
    )
    def gather_sum(ids_hbm, pids_hbm, wtab_hbm, ptab_hbm, out_hbm,
                   idx_v, pidx_v, wbuf, pbuf, obuf, sem_w, sem_p, sem_o):
        wid = lax.axis_index("s") * NUM_CORES + lax.axis_index("c")
        base = wid * tok_per_w
        pltpu.sync_copy(ids_hbm.at[pl.ds(base, tok_per_w)], idx_v)
        pltpu.sync_copy(pids_hbm.at[pl.ds(base, tok_per_w)], pidx_v)

        def fire_gathers(c, b):
            off = c * CHUNK
            pltpu.async_copy(
                wtab_hbm.at[idx_v.at[pl.ds(off, CHUNK)]], wbuf.at[b],
                sem_w[b])
            pltpu.async_copy(
                ptab_hbm.at[pidx_v.at[pl.ds(off, CHUNK)]], pbuf.at[b],
                sem_p[b])

        for b in range(NBUF):
            fire_gathers(b, b)

        def outer_body(o, carry):
            for b in range(NBUF):
                c = o * NBUF + b
                pltpu.make_async_copy(
                    wtab_hbm.at[idx_v.at[pl.ds(0, CHUNK)]], wbuf.at[b],
                    sem_w[b]).wait()
                pltpu.make_async_copy(
                    ptab_hbm.at[pidx_v.at[pl.ds(0, CHUNK)]], pbuf.at[b],
                    sem_p[b]).wait()
                @pl.when(o > 0)
                def _():
                    pltpu.make_async_copy(
                        obuf.at[b], out_hbm.at[pl.ds(0, CHUNK)],
                        sem_o[b]).wait()

                def row_body(r, carry2):
                    for v in range(VECS_PER_ROW):
                        sl = pl.ds(v * LANES, LANES)
                        obuf[b, r, sl] = wbuf[b, r, sl] + pbuf[b, r, sl]
                    return carry2

                lax.fori_loop(0, CHUNK, row_body, 0, unroll=False)
                pltpu.async_copy(
                    obuf.at[b], out_hbm.at[pl.ds(base + c * CHUNK, CHUNK)],
                    sem_o[b])
                @pl.when(c + NBUF < n_chunks)
                def _():
                    fire_gathers(c + NBUF, b)
            return carry

        lax.fori_loop(0, n_outer, outer_body, 0, unroll=False)
        for b in range(NBUF):
            pltpu.make_async_copy(
                obuf.at[b], out_hbm.at[pl.ds(0, CHUNK)], sem_o[b]).wait()

    return gather_sum


def _ln_body(x_ref, t_ref, g_ref, b_ref, o_ref):
    e = x_ref[...] + t_ref[...]
    mu = jnp.mean(e, axis=-1, keepdims=True)
    d = e - mu
    var = jnp.mean(d * d, axis=-1, keepdims=True)
    o_ref[...] = d * lax.rsqrt(var + EPS) * g_ref[...] + b_ref[...]


def _layernorm(summed, type_row, gamma, beta, blk):
    n = summed.shape[0]
    return pl.pallas_call(
        _ln_body,
        grid=(n // blk,),
        in_specs=[
            pl.BlockSpec((blk, HID), lambda i: (i, 0)),
            pl.BlockSpec((1, HID), lambda i: (0, 0)),
            pl.BlockSpec((1, HID), lambda i: (0, 0)),
            pl.BlockSpec((1, HID), lambda i: (0, 0)),
        ],
        out_specs=pl.BlockSpec((blk, HID), lambda i: (i, 0)),
        out_shape=jax.ShapeDtypeStruct((n, HID), jnp.float32),
    )(summed, type_row, gamma, beta)


def kernel(input_ids, position_ids, word_table, pos_table, type_table,
           gamma, beta):
    b, s = input_ids.shape
    n = b * s
    ids = input_ids.reshape(n)
    pids = position_ids.reshape(n)
    summed = _make_gather_sum(n)(ids, pids, word_table, pos_table)
    out = _layernorm(
        summed,
        type_table[0:1, :],
        gamma.reshape(1, HID),
        beta.reshape(1, HID),
        blk=512,
    )
    return out.reshape(b, s, HID)
